# baseline (speedup 1.0000x reference)
import functools

import jax
import jax.numpy as jnp
from jax import lax
from jax.experimental import pallas as pl
from jax.experimental.pallas import tpu as pltpu
from jax.experimental.pallas import tpu_sc as plsc

N = 10000
E = 160000
D = 256
HALF = 128

NC = 2
NS = 16
CHUNK = 125
BLK = 8
NBLK = E // (NS * BLK * CHUNK)
ZB = 80
NZCH = N // ZB
R = 1000


def _sc_agg_body(x2, src4, dst4, zrows, out,
                 srcA, dstA, srcB, dstB, rows0, rows1, agg_s,
                 gs0, gs1, isem, ss1):
    c = lax.axis_index("c")
    s = lax.axis_index("s")
    tab = x2.at[c]
    rows0z = rows0.at[pl.ds(0, ZB)]
    rows1z = rows1.at[pl.ds(0, ZB)]

    nz = jnp.where(s < NZCH - 7 * NS, 8, 7)
    pltpu.sync_copy(zrows, rows0z)

    def zfire(i, carry):
        pltpu.async_copy(rows0z, agg_s.at[pl.ds((s + NS * i) * ZB, ZB)], gs0)
        return carry

    lax.fori_loop(0, nz, zfire, 0)

    def zdrain(i, carry):
        pltpu.make_async_copy(rows0z, agg_s.at[pl.ds(s * ZB, ZB)], gs0).wait()
        return carry

    lax.fori_loop(0, nz, zdrain, 0)
    plsc.subcore_barrier()

    def emit_block(k, cur_src, cur_dst, nxt_src, nxt_dst, have_next, first):
        @pl.when(have_next)
        def _():
            pltpu.async_copy(src4.at[s, k + 1], nxt_src, isem)
            pltpu.async_copy(dst4.at[s, k + 1], nxt_dst, isem)

        for p in range(BLK // 2):
            i0 = 2 * p
            i1 = i0 + 1
            if not (first and p == 0):
                pltpu.make_async_copy(rows1, agg_s.at[cur_dst.at[i1]], ss1).wait()
            pltpu.async_copy(tab.at[cur_src.at[i1]], rows1, gs1)
            pltpu.make_async_copy(tab.at[cur_src.at[i0]], rows0, gs0).wait()
            pltpu.sync_copy(rows0, agg_s.at[cur_dst.at[i0]], add=True)
            if i0 + 2 < BLK:
                pltpu.async_copy(tab.at[cur_src.at[i0 + 2]], rows0, gs0)
            else:
                @pl.when(have_next)
                def _():
                    pltpu.make_async_copy(src4.at[s, k], nxt_src, isem).wait()
                    pltpu.make_async_copy(dst4.at[s, k], nxt_dst, isem).wait()
                    pltpu.async_copy(tab.at[nxt_src.at[0]], rows0, gs0)

            pltpu.make_async_copy(tab.at[cur_src.at[i1]], rows1, gs1).wait()
            pltpu.async_copy(rows1, agg_s.at[cur_dst.at[i1]], ss1, add=True)

    pltpu.sync_copy(src4.at[s, 0], srcA)
    pltpu.sync_copy(dst4.at[s, 0], dstA)
    pltpu.async_copy(tab.at[srcA.at[0]], rows0, gs0)

    emit_block(0, srcA, dstA, srcB, dstB, True, True)
    emit_block(1, srcB, dstB, srcA, dstA, True, False)

    def blkpair(m, carry):
        k0 = 2 * m + 2
        emit_block(k0, srcA, dstA, srcB, dstB, True, False)
        emit_block(k0 + 1, srcB, dstB, srcA, dstA, k0 + 2 < NBLK, False)
        return carry

    lax.fori_loop(0, (NBLK - 2) // 2, blkpair, 0)
    pltpu.make_async_copy(rows1, agg_s.at[dstA.at[BLK - 1]], ss1).wait()
    plsc.subcore_barrier()

    outc = out.at[c]

    def wb(q, carry):
        i0 = 2 * q
        i1 = i0 + 1
        off0 = (s + NS * i0) * ZB
        off1 = (s + NS * i1) * ZB

        @pl.when(q > 0)
        def _():
            pltpu.make_async_copy(rows0z, outc.at[pl.ds(s * ZB, ZB)], gs0).wait()
            pltpu.make_async_copy(rows1z, outc.at[pl.ds(s * ZB, ZB)], gs1).wait()

        pltpu.sync_copy(agg_s.at[pl.ds(off0, ZB)], rows0z)
        pltpu.async_copy(rows0z, outc.at[pl.ds(off0, ZB)], gs0)

        @pl.when(i1 < nz)
        def _():
            pltpu.sync_copy(agg_s.at[pl.ds(off1, ZB)], rows1z)
            pltpu.async_copy(rows1z, outc.at[pl.ds(off1, ZB)], gs1)

        return carry

    lax.fori_loop(0, 4, wb, 0)
    pltpu.make_async_copy(rows0z, outc.at[pl.ds(s * ZB, ZB)], gs0).wait()

    @pl.when(nz == 8)
    def _():
        pltpu.make_async_copy(rows1z, outc.at[pl.ds(s * ZB, ZB)], gs1).wait()


@functools.cache
def _sc_agg():
    return pl.kernel(
        _sc_agg_body,
        mesh=plsc.VectorSubcoreMesh(core_axis_name="c", subcore_axis_name="s"),
        out_type=jax.ShapeDtypeStruct((NC, N, HALF), jnp.float32),
        scratch_types=[
            pltpu.VMEM((BLK, CHUNK), jnp.int32),
            pltpu.VMEM((BLK, CHUNK), jnp.int32),
            pltpu.VMEM((BLK, CHUNK), jnp.int32),
            pltpu.VMEM((BLK, CHUNK), jnp.int32),
            pltpu.VMEM((CHUNK, HALF), jnp.float32),
            pltpu.VMEM((CHUNK, HALF), jnp.float32),
            pltpu.VMEM_SHARED((N, HALF), jnp.float32),
            pltpu.SemaphoreType.DMA,
            pltpu.SemaphoreType.DMA,
            pltpu.SemaphoreType.DMA,
            pltpu.SemaphoreType.DMA,
        ],
    )


def _mm_bf16(lhs, rhs_ref):
    dn = (((1,), (1,)), ((), ()))
    return lax.dot_general(lhs.astype(jnp.bfloat16),
                           rhs_ref[...].astype(jnp.bfloat16), dn,
                           preferred_element_type=jnp.float32)


def _tc_layer_body(agg_ref, x_ref, wrel_ref, b_ref, wroot_ref, out_ref):
    a = jnp.concatenate([agg_ref[0], agg_ref[1]], axis=1)
    xx = jnp.concatenate([x_ref[0], x_ref[1]], axis=1)
    acc = _mm_bf16(a, wrel_ref) + _mm_bf16(xx, wroot_ref)
    j = pl.program_id(0)
    acc = acc + b_ref[pl.ds(j, 1), :]
    out_ref[0] = jnp.maximum(acc, 0.0)


_tc_layer = pl.pallas_call(
    _tc_layer_body,
    grid=(2, N // R),
    in_specs=[
        pl.BlockSpec((2, R, HALF), lambda j, i: (0, i, 0)),
        pl.BlockSpec((2, R, HALF), lambda j, i: (0, i, 0)),
        pl.BlockSpec((HALF, D), lambda j, i: (j, 0)),
        pl.BlockSpec((NC, HALF), lambda j, i: (0, 0)),
        pl.BlockSpec((HALF, D), lambda j, i: (j, 0)),
    ],
    out_specs=pl.BlockSpec((1, R, HALF), lambda j, i: (j, i, 0)),
    out_shape=jax.ShapeDtypeStruct((NC, N, HALF), jnp.float32),
)


def _tc_final_body(agg_ref, x_ref, wrel_ref, b_ref, wroot_ref, out_ref):
    a = jnp.concatenate([agg_ref[0], agg_ref[1]], axis=1)
    xx = jnp.concatenate([x_ref[0], x_ref[1]], axis=1)
    acc = _mm_bf16(a, wrel_ref) + _mm_bf16(xx, wroot_ref)
    out_ref[...] = acc + b_ref[...][None, :]


_tc_final = pl.pallas_call(
    _tc_final_body,
    grid=(N // R,),
    in_specs=[
        pl.BlockSpec((2, R, HALF), lambda i: (0, i, 0)),
        pl.BlockSpec((2, R, HALF), lambda i: (0, i, 0)),
        pl.BlockSpec((D, D), lambda i: (0, 0)),
        pl.BlockSpec((D,), lambda i: (0,)),
        pl.BlockSpec((D, D), lambda i: (0, 0)),
    ],
    out_specs=pl.BlockSpec((R, D), lambda i: (i, 0)),
    out_shape=jax.ShapeDtypeStruct((N, D), jnp.float32),
)


def kernel(h, edge_index, Wrel0, brel0, Wroot0, Wrel1, brel1, Wroot1,
           Wrel2, brel2, Wroot2):
    src4 = edge_index[0].reshape(NS, NBLK, BLK, CHUNK)
    dst4 = edge_index[1].reshape(NS, NBLK, BLK, CHUNK)
    zrows = jnp.zeros((ZB, HALF), jnp.float32)

    x = h.reshape(N, NC, HALF).transpose(1, 0, 2)

    for Wrel, brel, Wroot in ((Wrel0, brel0, Wroot0), (Wrel1, brel1, Wroot1)):
        agg = _sc_agg()(x, src4, dst4, zrows)
        x = _tc_layer(agg, x, Wrel, brel.reshape(NC, HALF), Wroot)

    agg = _sc_agg()(x, src4, dst4, zrows)
    return _tc_final(agg, x, Wrel2, brel2, Wroot2)


# speedup vs baseline: 1.0072x; 1.0072x over previous
import functools

import jax
import jax.numpy as jnp
from jax import lax
from jax.experimental import pallas as pl
from jax.experimental.pallas import tpu as pltpu
from jax.experimental.pallas import tpu_sc as plsc

N = 10000
E = 160000
D = 256
HALF = 128

NC = 2
NS = 16
CHUNK = 125
BLK = 8
NBLK = E // (NS * BLK * CHUNK)
ZB = 80
NZCH = N // ZB
R = 1000


def _sc_agg_body(tab, src5, dst4, zrows, out,
                 srcA, dstA, srcB, dstB, rows0, rows1, agg_s,
                 gs0, gs1, isem, ss1):
    c = lax.axis_index("c")
    s = lax.axis_index("s")
    rows0z = rows0.at[pl.ds(0, ZB)]
    rows1z = rows1.at[pl.ds(0, ZB)]

    nz = jnp.where(s < NZCH - 7 * NS, 8, 7)
    pltpu.sync_copy(zrows, rows0z)

    def zfire(i, carry):
        pltpu.async_copy(rows0z, agg_s.at[pl.ds((s + NS * i) * ZB, ZB)], gs0)
        return carry

    lax.fori_loop(0, nz, zfire, 0)

    def zdrain(i, carry):
        pltpu.make_async_copy(rows0z, agg_s.at[pl.ds(s * ZB, ZB)], gs0).wait()
        return carry

    lax.fori_loop(0, nz, zdrain, 0)
    plsc.subcore_barrier()

    def emit_block(k, cur_src, cur_dst, nxt_src, nxt_dst, have_next, first):
        @pl.when(have_next)
        def _():
            pltpu.async_copy(src5.at[c, s, k + 1], nxt_src, isem)
            pltpu.async_copy(dst4.at[s, k + 1], nxt_dst, isem)

        for p in range(BLK // 2):
            i0 = 2 * p
            i1 = i0 + 1
            if not (first and p == 0):
                pltpu.make_async_copy(rows1, agg_s.at[cur_dst.at[i1]], ss1).wait()
            pltpu.async_copy(tab.at[cur_src.at[i1]], rows1, gs1)
            pltpu.make_async_copy(tab.at[cur_src.at[i0]], rows0, gs0).wait()
            pltpu.sync_copy(rows0, agg_s.at[cur_dst.at[i0]], add=True)
            if i0 + 2 < BLK:
                pltpu.async_copy(tab.at[cur_src.at[i0 + 2]], rows0, gs0)
            else:
                @pl.when(have_next)
                def _():
                    pltpu.make_async_copy(src5.at[c, s, k], nxt_src, isem).wait()
                    pltpu.make_async_copy(dst4.at[s, k], nxt_dst, isem).wait()
                    pltpu.async_copy(tab.at[nxt_src.at[0]], rows0, gs0)

            pltpu.make_async_copy(tab.at[cur_src.at[i1]], rows1, gs1).wait()
            pltpu.async_copy(rows1, agg_s.at[cur_dst.at[i1]], ss1, add=True)

    pltpu.sync_copy(src5.at[c, s, 0], srcA)
    pltpu.sync_copy(dst4.at[s, 0], dstA)
    pltpu.async_copy(tab.at[srcA.at[0]], rows0, gs0)

    emit_block(0, srcA, dstA, srcB, dstB, True, True)
    emit_block(1, srcB, dstB, srcA, dstA, True, False)

    def blkpair(m, carry):
        k0 = 2 * m + 2
        emit_block(k0, srcA, dstA, srcB, dstB, True, False)
        emit_block(k0 + 1, srcB, dstB, srcA, dstA, k0 + 2 < NBLK, False)
        return carry

    lax.fori_loop(0, (NBLK - 2) // 2, blkpair, 0)
    pltpu.make_async_copy(rows1, agg_s.at[dstA.at[BLK - 1]], ss1).wait()
    plsc.subcore_barrier()

    outc = out.at[c]

    def wb(q, carry):
        i0 = 2 * q
        i1 = i0 + 1
        off0 = (s + NS * i0) * ZB
        off1 = (s + NS * i1) * ZB

        @pl.when(q > 0)
        def _():
            pltpu.make_async_copy(rows0z, outc.at[pl.ds(s * ZB, ZB)], gs0).wait()
            pltpu.make_async_copy(rows1z, outc.at[pl.ds(s * ZB, ZB)], gs1).wait()

        pltpu.sync_copy(agg_s.at[pl.ds(off0, ZB)], rows0z)
        pltpu.async_copy(rows0z, outc.at[pl.ds(off0, ZB)], gs0)

        @pl.when(i1 < nz)
        def _():
            pltpu.sync_copy(agg_s.at[pl.ds(off1, ZB)], rows1z)
            pltpu.async_copy(rows1z, outc.at[pl.ds(off1, ZB)], gs1)

        return carry

    lax.fori_loop(0, 4, wb, 0)
    pltpu.make_async_copy(rows0z, outc.at[pl.ds(s * ZB, ZB)], gs0).wait()

    @pl.when(nz == 8)
    def _():
        pltpu.make_async_copy(rows1z, outc.at[pl.ds(s * ZB, ZB)], gs1).wait()


@functools.cache
def _sc_agg():
    return pl.kernel(
        _sc_agg_body,
        mesh=plsc.VectorSubcoreMesh(core_axis_name="c", subcore_axis_name="s"),
        out_type=jax.ShapeDtypeStruct((NC, N, HALF), jnp.float32),
        scratch_types=[
            pltpu.VMEM((BLK, CHUNK), jnp.int32),
            pltpu.VMEM((BLK, CHUNK), jnp.int32),
            pltpu.VMEM((BLK, CHUNK), jnp.int32),
            pltpu.VMEM((BLK, CHUNK), jnp.int32),
            pltpu.VMEM((CHUNK, HALF), jnp.float32),
            pltpu.VMEM((CHUNK, HALF), jnp.float32),
            pltpu.VMEM_SHARED((N, HALF), jnp.float32),
            pltpu.SemaphoreType.DMA,
            pltpu.SemaphoreType.DMA,
            pltpu.SemaphoreType.DMA,
            pltpu.SemaphoreType.DMA,
        ],
    )


def _mm_bf16(lhs, rhs_ref):
    dn = (((1,), (1,)), ((), ()))
    return lax.dot_general(lhs.astype(jnp.bfloat16),
                           rhs_ref[...].astype(jnp.bfloat16), dn,
                           preferred_element_type=jnp.float32)


def _make_tc_layer_body(split_x):
    def body(agg_ref, x_ref, wrel_ref, b_ref, wroot_ref, out_ref):
        a = jnp.concatenate([agg_ref[0], agg_ref[1]], axis=1)
        if split_x:
            xx = jnp.concatenate([x_ref[0], x_ref[1]], axis=1)
        else:
            xx = x_ref[...]
        acc = _mm_bf16(a, wrel_ref) + _mm_bf16(xx, wroot_ref)
        j = pl.program_id(0)
        acc = acc + b_ref[pl.ds(j, 1), :]
        out_ref[0] = jnp.maximum(acc, 0.0)
    return body


def _make_tc_layer(split_x):
    return pl.pallas_call(
        _make_tc_layer_body(split_x),
        grid=(2, N // R),
        in_specs=[
            pl.BlockSpec((2, R, HALF), lambda j, i: (0, i, 0)),
            (pl.BlockSpec((2, R, HALF), lambda j, i: (0, i, 0)) if split_x
             else pl.BlockSpec((R, D), lambda j, i: (i, 0))),
            pl.BlockSpec((HALF, D), lambda j, i: (j, 0)),
            pl.BlockSpec((NC, HALF), lambda j, i: (0, 0)),
            pl.BlockSpec((HALF, D), lambda j, i: (j, 0)),
        ],
        out_specs=pl.BlockSpec((1, R, HALF), lambda j, i: (j, i, 0)),
        out_shape=jax.ShapeDtypeStruct((NC, N, HALF), jnp.float32),
    )


_tc_layer0 = _make_tc_layer(False)
_tc_layer = _make_tc_layer(True)


def _tc_final_body(agg_ref, x_ref, wrel_ref, b_ref, wroot_ref, out_ref):
    a = jnp.concatenate([agg_ref[0], agg_ref[1]], axis=1)
    xx = jnp.concatenate([x_ref[0], x_ref[1]], axis=1)
    acc = _mm_bf16(a, wrel_ref) + _mm_bf16(xx, wroot_ref)
    out_ref[...] = acc + b_ref[...][None, :]


_tc_final = pl.pallas_call(
    _tc_final_body,
    grid=(N // R,),
    in_specs=[
        pl.BlockSpec((2, R, HALF), lambda i: (0, i, 0)),
        pl.BlockSpec((2, R, HALF), lambda i: (0, i, 0)),
        pl.BlockSpec((D, D), lambda i: (0, 0)),
        pl.BlockSpec((D,), lambda i: (0,)),
        pl.BlockSpec((D, D), lambda i: (0, 0)),
    ],
    out_specs=pl.BlockSpec((R, D), lambda i: (i, 0)),
    out_shape=jax.ShapeDtypeStruct((N, D), jnp.float32),
)


def kernel(h, edge_index, Wrel0, brel0, Wroot0, Wrel1, brel1, Wroot1,
           Wrel2, brel2, Wroot2):
    src1 = edge_index[0].reshape(1, NS, NBLK, BLK, CHUNK)
    dst4 = edge_index[1].reshape(NS, NBLK, BLK, CHUNK)
    coreix = jnp.arange(NC, dtype=jnp.int32).reshape(NC, 1, 1, 1, 1)
    idx_h = src1 * 2 + coreix
    idx_x = src1 + coreix * N
    zrows = jnp.zeros((ZB, HALF), jnp.float32)

    agg = _sc_agg()(h.reshape(NC * N, HALF), idx_h, dst4, zrows)
    x = _tc_layer0(agg, h, Wrel0, brel0.reshape(NC, HALF), Wroot0)

    agg = _sc_agg()(x.reshape(NC * N, HALF), idx_x, dst4, zrows)
    x = _tc_layer(agg, x, Wrel1, brel1.reshape(NC, HALF), Wroot1)

    agg = _sc_agg()(x.reshape(NC * N, HALF), idx_x, dst4, zrows)
    return _tc_final(agg, x, Wrel2, brel2, Wroot2)
